# baseline (speedup 1.0000x reference)
import functools

import jax
import jax.numpy as jnp
from jax import lax
from jax.experimental import pallas as pl
from jax.experimental.pallas import tpu as pltpu
from jax.experimental.pallas import tpu_sc as plsc

BATCH = 4096
SEQ_LEN = 50
NUM_RESIDUES = 1000
EMBED_DIM = 64

NUM_WORKERS = 32
TOTAL = BATCH * SEQ_LEN
PER_W = TOTAL // NUM_WORKERS
NBUF = 2
GROUPS = 16
GR = PER_W // GROUPS
TABLE_WORDS = NUM_RESIDUES * EMBED_DIM


def _sc_gather(idx_flat, table_flat):
    mesh = plsc.VectorSubcoreMesh(core_axis_name="c", subcore_axis_name="s")

    @functools.partial(
        pl.kernel,
        mesh=mesh,
        compiler_params=pltpu.CompilerParams(
            use_tc_tiling_on_sc=False, needs_layout_passes=False
        ),
        out_type=jax.ShapeDtypeStruct((TOTAL * EMBED_DIM,), jnp.float32),
        scratch_types=[
            pltpu.VMEM((TABLE_WORDS,), jnp.float32),
            pltpu.VMEM((PER_W,), jnp.int32),
            pltpu.VMEM((NBUF, GR * EMBED_DIM), jnp.float32),
        ]
        + [pltpu.SemaphoreType.DMA] * NBUF,
    )
    def k(idx_hbm, table_hbm, out_hbm, table_v, idx_v, obuf_v, *osem):
        wid = lax.axis_index("s") * 2 + lax.axis_index("c")
        base = wid * PER_W
        pltpu.sync_copy(idx_hbm.at[pl.ds(base, PER_W)], idx_v)
        pltpu.sync_copy(table_hbm, table_v)

        iota = lax.iota(jnp.int32, 16)
        obase0 = iota * EMBED_DIM

        def fill(g, b):
            def body(j, carry):
                rows = idx_v[pl.ds(g * GR + j * 16, 16)]
                gbase = rows * EMBED_DIM
                obase = obase0 + j * (16 * EMBED_DIM)
                for c in range(EMBED_DIM):
                    v = plsc.load_gather(table_v, [gbase + c])
                    plsc.store_scatter(obuf_v.at[b], [obase + c], v)
                return carry

            lax.fori_loop(0, GR // 16, body, 0)

        def fire_scatter(g, b):
            pltpu.async_copy(
                obuf_v.at[b],
                out_hbm.at[pl.ds((base + g * GR) * EMBED_DIM, GR * EMBED_DIM)],
                osem[b],
            )

        def wait_scatter(b):
            pltpu.make_async_copy(
                obuf_v.at[b],
                out_hbm.at[pl.ds(0, GR * EMBED_DIM)],
                osem[b],
            ).wait()

        for g in range(GROUPS):
            b = g % NBUF
            if g >= NBUF:
                wait_scatter(b)
            fill(g, b)
            fire_scatter(g, b)
        for b in range(NBUF):
            wait_scatter(b)

    return k(idx_flat, table_flat)


def kernel(indices, embeddings):
    idx = jnp.clip(indices, 0, NUM_RESIDUES - 1)
    out = _sc_gather(idx.reshape(TOTAL), embeddings.reshape(TABLE_WORDS))
    return out.reshape(BATCH, SEQ_LEN, EMBED_DIM)


# speedup vs baseline: 1.3755x; 1.3755x over previous
import functools

import jax
import jax.numpy as jnp
from jax import lax
from jax.experimental import pallas as pl
from jax.experimental.pallas import tpu as pltpu
from jax.experimental.pallas import tpu_sc as plsc

BATCH = 4096
SEQ_LEN = 50
NUM_RESIDUES = 1000
EMBED_DIM = 64

NUM_WORKERS = 32
TOTAL = BATCH * SEQ_LEN
PER_W = TOTAL // NUM_WORKERS
NBUF = 2
GROUPS = 20
GR = PER_W // GROUPS
TABLE_WORDS = NUM_RESIDUES * EMBED_DIM


def _sc_gather(idx_flat, table_flat):
    mesh = plsc.VectorSubcoreMesh(core_axis_name="c", subcore_axis_name="s")

    @functools.partial(
        pl.kernel,
        mesh=mesh,
        compiler_params=pltpu.CompilerParams(
            use_tc_tiling_on_sc=False, needs_layout_passes=False
        ),
        out_type=jax.ShapeDtypeStruct((TOTAL * EMBED_DIM,), jnp.float32),
        scratch_types=[
            pltpu.VMEM((TABLE_WORDS,), jnp.float32),
            pltpu.VMEM((PER_W,), jnp.int32),
            pltpu.VMEM((NBUF, GR * EMBED_DIM), jnp.float32),
        ]
        + [pltpu.SemaphoreType.DMA] * NBUF,
    )
    def k(idx_hbm, table_hbm, out_hbm, table_v, idx_v, obuf_v, *osem):
        wid = lax.axis_index("s") * 2 + lax.axis_index("c")
        base = wid * PER_W
        pltpu.sync_copy(idx_hbm.at[pl.ds(base, PER_W)], idx_v)
        pltpu.sync_copy(table_hbm, table_v)

        iota = lax.iota(jnp.int32, 16)
        obase0 = iota * EMBED_DIM

        def fill(g, b):
            @plsc.parallel_loop(0, GR // 16, unroll=1)
            def body(j):
                rows = idx_v[pl.ds(g * GR + j * 16, 16)]
                gbase = rows * EMBED_DIM
                obase = obase0 + j * (16 * EMBED_DIM)
                for c in range(EMBED_DIM):
                    v = plsc.load_gather(table_v, [gbase + c])
                    plsc.store_scatter(obuf_v.at[b], [obase + c], v)

        def fire_scatter(g, b):
            pltpu.async_copy(
                obuf_v.at[b],
                out_hbm.at[pl.ds((base + g * GR) * EMBED_DIM, GR * EMBED_DIM)],
                osem[b],
            )

        def wait_scatter(b):
            pltpu.make_async_copy(
                obuf_v.at[b],
                out_hbm.at[pl.ds(0, GR * EMBED_DIM)],
                osem[b],
            ).wait()

        for b in range(NBUF):
            fill(b, b)
            fire_scatter(b, b)

        def outer(i, carry):
            for b in range(NBUF):
                g = i * NBUF + b
                wait_scatter(b)
                fill(g, b)
                fire_scatter(g, b)
            return carry

        lax.fori_loop(1, GROUPS // NBUF, outer, 0)
        for b in range(NBUF):
            wait_scatter(b)

    return k(idx_flat, table_flat)


def kernel(indices, embeddings):
    idx = jnp.clip(indices, 0, NUM_RESIDUES - 1)
    out = _sc_gather(idx.reshape(TOTAL), embeddings.reshape(TABLE_WORDS))
    return out.reshape(BATCH, SEQ_LEN, EMBED_DIM)


# speedup vs baseline: 2.7527x; 2.0012x over previous
import functools

import jax
import jax.numpy as jnp
from jax import lax
from jax.experimental import pallas as pl
from jax.experimental.pallas import tpu as pltpu
from jax.experimental.pallas import tpu_sc as plsc

BATCH = 4096
SEQ_LEN = 50
NUM_RESIDUES = 1000
EMBED_DIM = 64

NUM_WORKERS = 32
TOTAL = BATCH * SEQ_LEN
PER_W = TOTAL // NUM_WORKERS
NBUF = 4
GROUPS = 16
GR = PER_W // GROUPS


def _sc_gather(idx_flat, table):
    mesh = plsc.VectorSubcoreMesh(core_axis_name="c", subcore_axis_name="s")

    @functools.partial(
        pl.kernel,
        mesh=mesh,
        compiler_params=pltpu.CompilerParams(use_tc_tiling_on_sc=False),
        out_type=jax.ShapeDtypeStruct((TOTAL, EMBED_DIM), jnp.float32),
        scratch_types=[
            pltpu.VMEM((PER_W,), jnp.int32),
            pltpu.VMEM((NBUF, GR, EMBED_DIM), jnp.float32),
        ]
        + [pltpu.SemaphoreType.DMA] * (2 * NBUF),
    )
    def k(idx_hbm, table_hbm, out_hbm, idx_v, rows_v, *sems):
        gsem, osem = sems[:NBUF], sems[NBUF:]
        wid = lax.axis_index("s") * 2 + lax.axis_index("c")
        base = wid * PER_W
        pltpu.sync_copy(idx_hbm.at[pl.ds(base, PER_W)], idx_v)

        def fire_gather(g, b):
            pltpu.async_copy(
                table_hbm.at[idx_v.at[pl.ds(g * GR, GR)]], rows_v.at[b], gsem[b]
            )

        def wait_gather(b):
            pltpu.make_async_copy(
                table_hbm.at[pl.ds(0, GR)], rows_v.at[b], gsem[b]
            ).wait()

        def fire_scatter(g, b):
            pltpu.async_copy(
                rows_v.at[b],
                out_hbm.at[pl.ds(base + g * GR, GR)],
                osem[b],
            )

        def wait_scatter(b):
            pltpu.make_async_copy(
                rows_v.at[b],
                out_hbm.at[pl.ds(0, GR)],
                osem[b],
            ).wait()

        for b in range(NBUF):
            fire_gather(b, b)

        def body(i, carry):
            g0 = i * NBUF
            for b in range(NBUF):
                g = g0 + b
                wait_gather(b)
                fire_scatter(g, b)
                wait_scatter(b)
                fire_gather(g + NBUF, b)
            return carry

        lax.fori_loop(0, GROUPS // NBUF - 1, body, 0)

        for b in range(NBUF):
            g = GROUPS - NBUF + b
            wait_gather(b)
            fire_scatter(g, b)
        for b in range(NBUF):
            wait_scatter(b)

    return k(idx_flat, table)


def kernel(indices, embeddings):
    idx = jnp.clip(indices, 0, NUM_RESIDUES - 1)
    out = _sc_gather(idx.reshape(TOTAL), embeddings)
    return out.reshape(BATCH, SEQ_LEN, EMBED_DIM)
